# Initial kernel scaffold; baseline (speedup 1.0000x reference)
import functools

import jax
import jax.numpy as jnp
from jax import lax
from jax.experimental import pallas as pl
from jax.experimental.pallas import tpu as pltpu
from jax.experimental.pallas import tpu_sc as plsc


def _round_up(x, m):
    return (x + m - 1) // m * m


def _mm_bias_relu(x, w, b2):
    n, fd = x.shape
    h = w.shape[1]
    bn = 2000
    grid = n // bn

    def body(x_ref, w_ref, b_ref, o_ref):
        o_ref[...] = jnp.maximum(
            jnp.dot(x_ref[...], w_ref[...], preferred_element_type=jnp.float32)
            + b_ref[...],
            0.0,
        )

    return pl.pallas_call(
        body,
        grid=(grid,),
        in_specs=[
            pl.BlockSpec((bn, fd), lambda i: (i, 0)),
            pl.BlockSpec((fd, h), lambda i: (0, 0)),
            pl.BlockSpec((1, h), lambda i: (0, 0)),
        ],
        out_specs=pl.BlockSpec((bn, h), lambda i: (i, 0)),
        out_shape=jax.ShapeDtypeStruct((n, h), jnp.float32),
    )(x, w, b2)


def _mm2_bias_relu(a, nh, w, b2):
    n, h = a.shape

    def body(a_ref, nh_ref, w_ref, b_ref, o_ref):
        o_ref[...] = jnp.maximum(
            jnp.dot(a_ref[...] + nh_ref[...], w_ref[...],
                    preferred_element_type=jnp.float32)
            + b_ref[...],
            0.0,
        )

    bn = 2000
    return pl.pallas_call(
        body,
        grid=(n // bn,),
        in_specs=[
            pl.BlockSpec((bn, h), lambda i: (i, 0)),
            pl.BlockSpec((bn, h), lambda i: (i, 0)),
            pl.BlockSpec((h, h), lambda i: (0, 0)),
            pl.BlockSpec((1, h), lambda i: (0, 0)),
        ],
        out_specs=pl.BlockSpec((bn, h), lambda i: (i, 0)),
        out_shape=jax.ShapeDtypeStruct((n, h), jnp.float32),
    )(a, nh, w, b2)


def _mm2_colsum(a, nh, w, b2):
    n, h = a.shape

    def body(a_ref, nh_ref, w_ref, b_ref, o_ref):
        t = jnp.maximum(
            jnp.dot(a_ref[...] + nh_ref[...], w_ref[...],
                    preferred_element_type=jnp.float32)
            + b_ref[...],
            0.0,
        )
        s = jnp.sum(t, axis=0, keepdims=True)

        @pl.when(pl.program_id(0) == 0)
        def _():
            o_ref[...] = s

        @pl.when(pl.program_id(0) != 0)
        def _():
            o_ref[...] += s

    bn = 2000
    return pl.pallas_call(
        body,
        grid=(n // bn,),
        in_specs=[
            pl.BlockSpec((bn, h), lambda i: (i, 0)),
            pl.BlockSpec((bn, h), lambda i: (i, 0)),
            pl.BlockSpec((h, h), lambda i: (0, 0)),
            pl.BlockSpec((1, h), lambda i: (0, 0)),
        ],
        out_specs=pl.BlockSpec((1, h), lambda i: (0, 0)),
        out_shape=jax.ShapeDtypeStruct((1, h), jnp.float32),
    )(a, nh, w, b2)


def _final_proj(colsum, w_out, b2, n):
    def body(cs_ref, w_ref, b_ref, o_ref):
        o_ref[...] = (
            jnp.dot(cs_ref[...] * (1.0 / n), w_ref[...],
                    preferred_element_type=jnp.float32)
            + b_ref[...]
        )

    h, o = w_out.shape
    return pl.pallas_call(
        body,
        out_shape=jax.ShapeDtypeStruct((1, o), jnp.float32),
    )(colsum, w_out, b2)


def _make_segsum(n, h, e, nc, ns):
    half = _round_up(n, 2) // 2
    rpt = _round_up(-(-half // ns), 8)
    region = ns * rpt
    chunk = 128
    ept = _round_up(-(-e // ns), chunk)
    nchunk = ept // chunk
    zb = 56
    assert rpt % zb == 0

    mesh = plsc.VectorSubcoreMesh(core_axis_name="c", subcore_axis_name="s")

    @functools.partial(
        pl.kernel,
        out_type=jax.ShapeDtypeStruct((nc * region, h), jnp.float32),
        mesh=mesh,
        scratch_types=[
            pltpu.VMEM((chunk,), jnp.int32),
            pltpu.VMEM((chunk,), jnp.int32),
            pltpu.VMEM((1, chunk), jnp.int32),
            pltpu.VMEM((chunk, h), jnp.float32),
            pltpu.VMEM((zb, h), jnp.float32),
            pltpu.VMEM_SHARED((region + ns, h), jnp.float32),
            pltpu.SemaphoreType.DMA,
        ],
    )
    def segsum(rows_hbm, src_hbm, dst_hbm, out_hbm,
               src_v, dst_v, idx_v, rows_v, zero_v, acc, sem):
        c = lax.axis_index("c")
        s = lax.axis_index("s")
        base = c * half

        def zrow(i, carry):
            for j in range(h // 16):
                zero_v[i, pl.ds(j * 16, 16)] = jnp.zeros((16,), jnp.float32)
            return carry

        lax.fori_loop(0, zb, zrow, 0)

        def zcp(k, carry):
            pltpu.sync_copy(zero_v, acc.at[pl.ds(s * rpt + k * zb, zb)])
            return carry

        lax.fori_loop(0, rpt // zb, zcp, 0)
        plsc.subcore_barrier()

        dummy = region + s

        def chunk_body(j, carry):
            off = s * ept + j * chunk
            pltpu.sync_copy(src_hbm.at[pl.ds(off, chunk)], src_v)
            pltpu.sync_copy(dst_hbm.at[pl.ds(off, chunk)], dst_v)
            for v in range(chunk // 16):
                d = dst_v[pl.ds(v * 16, 16)]
                inr = (d >= base) & (d < base + half)
                idx_v[0, pl.ds(v * 16, 16)] = jnp.where(inr, d - base, dummy)
            pltpu.async_copy(rows_hbm.at[src_v], rows_v, sem).wait()
            pltpu.sync_copy(rows_v, acc.at[idx_v.at[0]], add=True)
            return carry

        lax.fori_loop(0, nchunk, chunk_body, 0)
        plsc.subcore_barrier()
        pltpu.sync_copy(acc.at[pl.ds(s * rpt, rpt)],
                        out_hbm.at[pl.ds(c * region + s * rpt, rpt)])

    return segsum, region, half, ept * ns


def kernel(nodes, edges, features, W_node, b_node, W_conv1, b_conv1, W_out, b_out):
    n, fd = features.shape
    h = W_node.shape[1]
    o = W_out.shape[1]
    e = edges.shape[0]

    nc, ns = 2, 16
    segsum, region, half, e_pad = _make_segsum(n, h, e, nc, ns)

    src_p = jnp.concatenate(
        [edges[:, 0], jnp.zeros((e_pad - e,), jnp.int32)])
    dst_p = jnp.concatenate(
        [edges[:, 1], jnp.full((e_pad - e,), n, jnp.int32)])

    b_node2 = b_node.reshape(1, h)
    b_conv2 = b_conv1.reshape(1, h)
    b_out2 = b_out.reshape(1, o)

    hid = _mm_bias_relu(features, W_node, b_node2)

    out = segsum(hid, src_p, dst_p)
    nh = jnp.concatenate([out[:half], out[region:region + (n - half)]], axis=0)
    hid = _mm2_bias_relu(hid, nh, W_conv1, b_conv2)

    out = segsum(hid, src_p, dst_p)
    nh = jnp.concatenate([out[:half], out[region:region + (n - half)]], axis=0)
    colsum = _mm2_colsum(hid, nh, W_conv1, b_conv2)

    metrics = _final_proj(colsum, W_out, b_out2, n)
    return metrics.reshape(o)


# speedup vs baseline: 3.5901x; 3.5901x over previous
import functools

import jax
import jax.numpy as jnp
from jax import lax
from jax.experimental import pallas as pl
from jax.experimental.pallas import tpu as pltpu
from jax.experimental.pallas import tpu_sc as plsc


def _round_up(x, m):
    return (x + m - 1) // m * m


def _mm_bias_relu(x, w, b2):
    n, fd = x.shape
    h = w.shape[1]
    bn = 2000
    grid = n // bn

    def body(x_ref, w_ref, b_ref, o_ref):
        o_ref[...] = jnp.maximum(
            jnp.dot(x_ref[...], w_ref[...], preferred_element_type=jnp.float32)
            + b_ref[...],
            0.0,
        )

    return pl.pallas_call(
        body,
        grid=(grid,),
        in_specs=[
            pl.BlockSpec((bn, fd), lambda i: (i, 0)),
            pl.BlockSpec((fd, h), lambda i: (0, 0)),
            pl.BlockSpec((1, h), lambda i: (0, 0)),
        ],
        out_specs=pl.BlockSpec((bn, h), lambda i: (i, 0)),
        out_shape=jax.ShapeDtypeStruct((n, h), jnp.float32),
    )(x, w, b2)


def _mm2_bias_relu(a, nh, w, b2):
    n, h = a.shape

    def body(a_ref, nh_ref, w_ref, b_ref, o_ref):
        o_ref[...] = jnp.maximum(
            jnp.dot(a_ref[...] + nh_ref[...], w_ref[...],
                    preferred_element_type=jnp.float32)
            + b_ref[...],
            0.0,
        )

    bn = 2000
    return pl.pallas_call(
        body,
        grid=(n // bn,),
        in_specs=[
            pl.BlockSpec((bn, h), lambda i: (i, 0)),
            pl.BlockSpec((bn, h), lambda i: (i, 0)),
            pl.BlockSpec((h, h), lambda i: (0, 0)),
            pl.BlockSpec((1, h), lambda i: (0, 0)),
        ],
        out_specs=pl.BlockSpec((bn, h), lambda i: (i, 0)),
        out_shape=jax.ShapeDtypeStruct((n, h), jnp.float32),
    )(a, nh, w, b2)


def _mm2_colsum(a, nh, w, b2):
    n, h = a.shape

    def body(a_ref, nh_ref, w_ref, b_ref, o_ref):
        t = jnp.maximum(
            jnp.dot(a_ref[...] + nh_ref[...], w_ref[...],
                    preferred_element_type=jnp.float32)
            + b_ref[...],
            0.0,
        )
        s = jnp.sum(t, axis=0, keepdims=True)

        @pl.when(pl.program_id(0) == 0)
        def _():
            o_ref[...] = s

        @pl.when(pl.program_id(0) != 0)
        def _():
            o_ref[...] += s

    bn = 2000
    return pl.pallas_call(
        body,
        grid=(n // bn,),
        in_specs=[
            pl.BlockSpec((bn, h), lambda i: (i, 0)),
            pl.BlockSpec((bn, h), lambda i: (i, 0)),
            pl.BlockSpec((h, h), lambda i: (0, 0)),
            pl.BlockSpec((1, h), lambda i: (0, 0)),
        ],
        out_specs=pl.BlockSpec((1, h), lambda i: (0, 0)),
        out_shape=jax.ShapeDtypeStruct((1, h), jnp.float32),
    )(a, nh, w, b2)


def _final_proj(colsum, w_out, b2, n):
    def body(cs_ref, w_ref, b_ref, o_ref):
        o_ref[...] = (
            jnp.dot(cs_ref[...] * (1.0 / n), w_ref[...],
                    preferred_element_type=jnp.float32)
            + b_ref[...]
        )

    h, o = w_out.shape
    return pl.pallas_call(
        body,
        out_shape=jax.ShapeDtypeStruct((1, o), jnp.float32),
    )(colsum, w_out, b2)


def _make_segsum(n, h, e, nc, ns):
    half = _round_up(n, 2) // 2
    rpt = _round_up(-(-half // ns), 8)
    region = ns * rpt
    chunk = 128
    ept = _round_up(-(-e // ns), chunk)
    nchunk = ept // chunk
    zb = 56
    assert rpt % zb == 0

    mesh = plsc.VectorSubcoreMesh(core_axis_name="c", subcore_axis_name="s")

    @functools.partial(
        pl.kernel,
        out_type=jax.ShapeDtypeStruct((nc * region, h), jnp.float32),
        mesh=mesh,
        compiler_params=pltpu.CompilerParams(use_tc_tiling_on_sc=False),
        scratch_types=[
            pltpu.VMEM((chunk,), jnp.int32),
            pltpu.VMEM((chunk,), jnp.int32),
            pltpu.VMEM((1, chunk), jnp.int32),
            pltpu.VMEM((chunk, h), jnp.float32),
            pltpu.VMEM((zb, h), jnp.float32),
            pltpu.VMEM_SHARED((region + ns, h), jnp.float32),
            pltpu.SemaphoreType.DMA,
        ],
    )
    def segsum(rows_hbm, src_hbm, dst_hbm, out_hbm,
               src_v, dst_v, idx_v, rows_v, zero_v, acc, sem):
        c = lax.axis_index("c")
        s = lax.axis_index("s")
        base = c * half

        def zrow(i, carry):
            for j in range(h // 16):
                zero_v[i, pl.ds(j * 16, 16)] = jnp.zeros((16,), jnp.float32)
            return carry

        lax.fori_loop(0, zb, zrow, 0)

        def zcp(k, carry):
            pltpu.sync_copy(zero_v, acc.at[pl.ds(s * rpt + k * zb, zb)])
            return carry

        lax.fori_loop(0, rpt // zb, zcp, 0)
        plsc.subcore_barrier()

        dummy = region + s

        def chunk_body(j, carry):
            off = s * ept + j * chunk
            pltpu.sync_copy(src_hbm.at[pl.ds(off, chunk)], src_v)
            pltpu.sync_copy(dst_hbm.at[pl.ds(off, chunk)], dst_v)
            for v in range(chunk // 16):
                d = dst_v[pl.ds(v * 16, 16)]
                inr = (d >= base) & (d < base + half)
                idx_v[0, pl.ds(v * 16, 16)] = jnp.where(inr, d - base, dummy)
            pltpu.async_copy(rows_hbm.at[src_v], rows_v, sem).wait()
            pltpu.sync_copy(rows_v, acc.at[idx_v.at[0]], add=True)
            return carry

        lax.fori_loop(0, nchunk, chunk_body, 0)
        plsc.subcore_barrier()
        pltpu.sync_copy(acc.at[pl.ds(s * rpt, rpt)],
                        out_hbm.at[pl.ds(c * region + s * rpt, rpt)])

    return segsum, region, half, ept * ns


def kernel(nodes, edges, features, W_node, b_node, W_conv1, b_conv1, W_out, b_out):
    n, fd = features.shape
    h = W_node.shape[1]
    o = W_out.shape[1]
    e = edges.shape[0]

    nc, ns = 2, 16
    segsum, region, half, e_pad = _make_segsum(n, h, e, nc, ns)

    src_p = jnp.concatenate(
        [edges[:, 0], jnp.zeros((e_pad - e,), jnp.int32)])
    dst_p = jnp.concatenate(
        [edges[:, 1], jnp.full((e_pad - e,), n, jnp.int32)])

    b_node2 = b_node.reshape(1, h)
    b_conv2 = b_conv1.reshape(1, h)
    b_out2 = b_out.reshape(1, o)

    hid = _mm_bias_relu(features, W_node, b_node2)

    out = segsum(hid, src_p, dst_p)
    nh = jnp.concatenate([out[:half], out[region:region + (n - half)]], axis=0)
    hid = _mm2_bias_relu(hid, nh, W_conv1, b_conv2)

    out = segsum(hid, src_p, dst_p)
    nh = jnp.concatenate([out[:half], out[region:region + (n - half)]], axis=0)
    colsum = _mm2_colsum(hid, nh, W_conv1, b_conv2)

    metrics = _final_proj(colsum, W_out, b_out2, n)
    return metrics.reshape(o)


# speedup vs baseline: 5.8488x; 1.6291x over previous
import functools

import jax
import jax.numpy as jnp
from jax import lax
from jax.experimental import pallas as pl
from jax.experimental.pallas import tpu as pltpu
from jax.experimental.pallas import tpu_sc as plsc


def _round_up(x, m):
    return (x + m - 1) // m * m


def _mm_bias_relu(x, w, b2):
    n, fd = x.shape
    h = w.shape[1]
    bn = 2000
    grid = n // bn

    def body(x_ref, w_ref, b_ref, o_ref):
        o_ref[...] = jnp.maximum(
            jnp.dot(x_ref[...], w_ref[...], preferred_element_type=jnp.float32)
            + b_ref[...],
            0.0,
        )

    return pl.pallas_call(
        body,
        grid=(grid,),
        in_specs=[
            pl.BlockSpec((bn, fd), lambda i: (i, 0)),
            pl.BlockSpec((fd, h), lambda i: (0, 0)),
            pl.BlockSpec((1, h), lambda i: (0, 0)),
        ],
        out_specs=pl.BlockSpec((bn, h), lambda i: (i, 0)),
        out_shape=jax.ShapeDtypeStruct((n, h), jnp.float32),
    )(x, w, b2)


def _mm2_bias_relu(a, nh, w, b2):
    n, h = a.shape

    def body(a_ref, nh_ref, w_ref, b_ref, o_ref):
        o_ref[...] = jnp.maximum(
            jnp.dot(a_ref[...] + nh_ref[...], w_ref[...],
                    preferred_element_type=jnp.float32)
            + b_ref[...],
            0.0,
        )

    bn = 2000
    return pl.pallas_call(
        body,
        grid=(n // bn,),
        in_specs=[
            pl.BlockSpec((bn, h), lambda i: (i, 0)),
            pl.BlockSpec((bn, h), lambda i: (i, 0)),
            pl.BlockSpec((h, h), lambda i: (0, 0)),
            pl.BlockSpec((1, h), lambda i: (0, 0)),
        ],
        out_specs=pl.BlockSpec((bn, h), lambda i: (i, 0)),
        out_shape=jax.ShapeDtypeStruct((n, h), jnp.float32),
    )(a, nh, w, b2)


def _mm2_colsum(a, nh, w, b2):
    n, h = a.shape

    def body(a_ref, nh_ref, w_ref, b_ref, o_ref):
        t = jnp.maximum(
            jnp.dot(a_ref[...] + nh_ref[...], w_ref[...],
                    preferred_element_type=jnp.float32)
            + b_ref[...],
            0.0,
        )
        s = jnp.sum(t, axis=0, keepdims=True)

        @pl.when(pl.program_id(0) == 0)
        def _():
            o_ref[...] = s

        @pl.when(pl.program_id(0) != 0)
        def _():
            o_ref[...] += s

    bn = 2000
    return pl.pallas_call(
        body,
        grid=(n // bn,),
        in_specs=[
            pl.BlockSpec((bn, h), lambda i: (i, 0)),
            pl.BlockSpec((bn, h), lambda i: (i, 0)),
            pl.BlockSpec((h, h), lambda i: (0, 0)),
            pl.BlockSpec((1, h), lambda i: (0, 0)),
        ],
        out_specs=pl.BlockSpec((1, h), lambda i: (0, 0)),
        out_shape=jax.ShapeDtypeStruct((1, h), jnp.float32),
    )(a, nh, w, b2)


def _final_proj(colsum, w_out, b2, n):
    def body(cs_ref, w_ref, b_ref, o_ref):
        o_ref[...] = (
            jnp.dot(cs_ref[...] * (1.0 / n), w_ref[...],
                    preferred_element_type=jnp.float32)
            + b_ref[...]
        )

    h, o = w_out.shape
    return pl.pallas_call(
        body,
        out_shape=jax.ShapeDtypeStruct((1, o), jnp.float32),
    )(colsum, w_out, b2)


def _make_segsum(n, h, e, nc, ns):
    half = _round_up(n, 2) // 2
    rpt = _round_up(-(-half // ns), 8)
    region = ns * rpt
    chunk = 128
    ept = _round_up(-(-e // ns), 2 * chunk)
    nchunk = ept // chunk
    zb = 56
    assert rpt % zb == 0 and nchunk % 2 == 0

    mesh = plsc.VectorSubcoreMesh(core_axis_name="c", subcore_axis_name="s")

    @functools.partial(
        pl.kernel,
        out_type=jax.ShapeDtypeStruct((nc * region, h), jnp.float32),
        mesh=mesh,
        compiler_params=pltpu.CompilerParams(use_tc_tiling_on_sc=False),
        scratch_types=[
            [pltpu.VMEM((chunk,), jnp.int32)] * 2,
            [pltpu.VMEM((chunk,), jnp.int32)] * 2,
            [pltpu.VMEM((1, chunk), jnp.int32)] * 2,
            [pltpu.VMEM((chunk, h), jnp.float32)] * 2,
            pltpu.VMEM((zb, h), jnp.float32),
            pltpu.VMEM_SHARED((region + ns, h), jnp.float32),
            [pltpu.SemaphoreType.DMA] * 2,
            [pltpu.SemaphoreType.DMA] * 2,
            [pltpu.SemaphoreType.DMA] * 2,
        ],
    )
    def segsum(rows_hbm, src_hbm, dst_hbm, out_hbm,
               src_s, dst_s, idx_v, rows_v, zero_v, acc,
               st_sem, g_sem, sc_sem):
        c = lax.axis_index("c")
        s = lax.axis_index("s")
        base = c * half

        def zrow(i, carry):
            for j in range(h // 16):
                zero_v[i, pl.ds(j * 16, 16)] = jnp.zeros((16,), jnp.float32)
            return carry

        lax.fori_loop(0, zb, zrow, 0)

        def zcp(k, carry):
            pltpu.sync_copy(zero_v, acc.at[pl.ds(s * rpt + k * zb, zb)])
            return carry

        lax.fori_loop(0, rpt // zb, zcp, 0)
        plsc.subcore_barrier()

        dummy = region + s

        def stage(j, p):
            off = s * ept + j * chunk
            pltpu.async_copy(src_hbm.at[pl.ds(off, chunk)], src_s[p], st_sem[p])
            pltpu.async_copy(dst_hbm.at[pl.ds(off, chunk)], dst_s[p], st_sem[p])

        def prep(j, p):
            off = s * ept + j * chunk
            pltpu.make_async_copy(
                src_hbm.at[pl.ds(off, chunk)], src_s[p], st_sem[p]).wait()
            pltpu.make_async_copy(
                dst_hbm.at[pl.ds(off, chunk)], dst_s[p], st_sem[p]).wait()
            for v in range(chunk // 16):
                d = dst_s[p][pl.ds(v * 16, 16)]
                inr = (d >= base) & (d < base + half)
                idx_v[p][0, pl.ds(v * 16, 16)] = jnp.where(inr, d - base, dummy)
            return pltpu.async_copy(rows_hbm.at[src_s[p]], rows_v[p], g_sem[p])

        def run_chunk(j, p):
            sd = pltpu.async_copy(rows_v[p], acc.at[idx_v[p].at[0]],
                                  sc_sem[p], add=True)
            gd = prep(jnp.minimum(j + 1, nchunk - 1), 1 - p)
            stage(jnp.minimum(j + 2, nchunk - 1), p)
            sd.wait()
            gd.wait()

        stage(0, 0)
        stage(1, 1)
        prep(0, 0).wait()

        def pair_body(i2, carry):
            run_chunk(2 * i2, 0)
            run_chunk(2 * i2 + 1, 1)
            return carry

        lax.fori_loop(0, nchunk // 2, pair_body, 0)
        plsc.subcore_barrier()
        pltpu.sync_copy(acc.at[pl.ds(s * rpt, rpt)],
                        out_hbm.at[pl.ds(c * region + s * rpt, rpt)])

    return segsum, region, half, ept * ns


def kernel(nodes, edges, features, W_node, b_node, W_conv1, b_conv1, W_out, b_out):
    n, fd = features.shape
    h = W_node.shape[1]
    o = W_out.shape[1]
    e = edges.shape[0]

    nc, ns = 2, 16
    segsum, region, half, e_pad = _make_segsum(n, h, e, nc, ns)

    src_p = jnp.concatenate(
        [edges[:, 0], jnp.zeros((e_pad - e,), jnp.int32)])
    dst_p = jnp.concatenate(
        [edges[:, 1], jnp.full((e_pad - e,), n, jnp.int32)])

    b_node2 = b_node.reshape(1, h)
    b_conv2 = b_conv1.reshape(1, h)
    b_out2 = b_out.reshape(1, o)

    hid = _mm_bias_relu(features, W_node, b_node2)

    out = segsum(hid, src_p, dst_p)
    nh = jnp.concatenate([out[:half], out[region:region + (n - half)]], axis=0)
    hid = _mm2_bias_relu(hid, nh, W_conv1, b_conv2)

    out = segsum(hid, src_p, dst_p)
    nh = jnp.concatenate([out[:half], out[region:region + (n - half)]], axis=0)
    colsum = _mm2_colsum(hid, nh, W_conv1, b_conv2)

    metrics = _final_proj(colsum, W_out, b_out2, n)
    return metrics.reshape(o)


# speedup vs baseline: 7.4052x; 1.2661x over previous
import functools

import jax
import jax.numpy as jnp
from jax import lax
from jax.experimental import pallas as pl
from jax.experimental.pallas import tpu as pltpu
from jax.experimental.pallas import tpu_sc as plsc


def _mm1_split(x, w, b2):
    n, fd = x.shape
    h = w.shape[1]
    hh = h // 2
    bn = 2000

    def body(x_ref, w_ref, b_ref, o_ref):
        t = jnp.maximum(
            jnp.dot(x_ref[...], w_ref[...], preferred_element_type=jnp.float32)
            + b_ref[...],
            0.0,
        )
        o_ref[0] = t[:, :hh]
        o_ref[1] = t[:, hh:]

    return pl.pallas_call(
        body,
        grid=(n // bn,),
        in_specs=[
            pl.BlockSpec((bn, fd), lambda i: (i, 0)),
            pl.BlockSpec((fd, h), lambda i: (0, 0)),
            pl.BlockSpec((1, h), lambda i: (0, 0)),
        ],
        out_specs=pl.BlockSpec((2, bn, hh), lambda i: (0, i, 0)),
        out_shape=jax.ShapeDtypeStruct((2, n, hh), jnp.float32),
    )(x, w, b2)


def _mm2_split(a, nh, w, b2):
    _, n, hh = a.shape
    h = 2 * hh
    bn = 2000

    def body(a_ref, nh_ref, w_ref, b_ref, o_ref):
        x = jnp.concatenate(
            [a_ref[0] + nh_ref[0], a_ref[1] + nh_ref[1]], axis=1)
        t = jnp.maximum(
            jnp.dot(x, w_ref[...], preferred_element_type=jnp.float32)
            + b_ref[...],
            0.0,
        )
        o_ref[0] = t[:, :hh]
        o_ref[1] = t[:, hh:]

    return pl.pallas_call(
        body,
        grid=(n // bn,),
        in_specs=[
            pl.BlockSpec((2, bn, hh), lambda i: (0, i, 0)),
            pl.BlockSpec((2, bn, hh), lambda i: (0, i, 0)),
            pl.BlockSpec((h, h), lambda i: (0, 0)),
            pl.BlockSpec((1, h), lambda i: (0, 0)),
        ],
        out_specs=pl.BlockSpec((2, bn, hh), lambda i: (0, i, 0)),
        out_shape=jax.ShapeDtypeStruct((2, n, hh), jnp.float32),
    )(a, nh, w, b2)


def _mm2_colsum(a, nh, w, b2):
    _, n, hh = a.shape
    h = 2 * hh
    bn = 2000

    def body(a_ref, nh_ref, w_ref, b_ref, o_ref):
        x = jnp.concatenate(
            [a_ref[0] + nh_ref[0], a_ref[1] + nh_ref[1]], axis=1)
        t = jnp.maximum(
            jnp.dot(x, w_ref[...], preferred_element_type=jnp.float32)
            + b_ref[...],
            0.0,
        )
        s = jnp.sum(t, axis=0, keepdims=True)

        @pl.when(pl.program_id(0) == 0)
        def _():
            o_ref[...] = s

        @pl.when(pl.program_id(0) != 0)
        def _():
            o_ref[...] += s

    return pl.pallas_call(
        body,
        grid=(n // bn,),
        in_specs=[
            pl.BlockSpec((2, bn, hh), lambda i: (0, i, 0)),
            pl.BlockSpec((2, bn, hh), lambda i: (0, i, 0)),
            pl.BlockSpec((h, h), lambda i: (0, 0)),
            pl.BlockSpec((1, h), lambda i: (0, 0)),
        ],
        out_specs=pl.BlockSpec((1, h), lambda i: (0, 0)),
        out_shape=jax.ShapeDtypeStruct((1, h), jnp.float32),
    )(a, nh, w, b2)


def _final_proj(colsum, w_out, b2, n):
    def body(cs_ref, w_ref, b_ref, o_ref):
        o_ref[...] = (
            jnp.dot(cs_ref[...] * (1.0 / n), w_ref[...],
                    preferred_element_type=jnp.float32)
            + b_ref[...]
        )

    h, o = w_out.shape
    return pl.pallas_call(
        body,
        out_shape=jax.ShapeDtypeStruct((1, o), jnp.float32),
    )(colsum, w_out, b2)


def _round_up(x, m):
    return (x + m - 1) // m * m


def _make_segsum(n, hh, e, nc, ns):
    assert n % ns == 0
    rpt = n // ns
    chunk = 128
    ept = _round_up(-(-e // ns), 2 * chunk)
    nchunk = ept // chunk
    zb = 125
    assert rpt % zb == 0

    mesh = plsc.VectorSubcoreMesh(core_axis_name="c", subcore_axis_name="s")

    @functools.partial(
        pl.kernel,
        out_type=jax.ShapeDtypeStruct((nc * n, hh), jnp.float32),
        mesh=mesh,
        compiler_params=pltpu.CompilerParams(use_tc_tiling_on_sc=False),
        scratch_types=[
            [pltpu.VMEM((chunk,), jnp.int32)] * 2,
            [pltpu.VMEM((1, chunk), jnp.int32)] * 2,
            [pltpu.VMEM((chunk, hh), jnp.float32)] * 2,
            pltpu.VMEM((zb, hh), jnp.float32),
            pltpu.VMEM_SHARED((n + 8, hh), jnp.float32),
            [pltpu.SemaphoreType.DMA] * 2,
            [pltpu.SemaphoreType.DMA] * 2,
            [pltpu.SemaphoreType.DMA] * 2,
        ],
    )
    def segsum(tab_hbm, src_hbm, dst_hbm, out_hbm,
               src_s, idx_v, rows_v, zero_v, acc,
               st_sem, g_sem, sc_sem):
        c = lax.axis_index("c")
        s = lax.axis_index("s")
        cbase = c * n

        def zrow(i, carry):
            for j in range(hh // 16):
                zero_v[i, pl.ds(j * 16, 16)] = jnp.zeros((16,), jnp.float32)
            return carry

        lax.fori_loop(0, zb, zrow, 0)

        def zcp(k, carry):
            pltpu.sync_copy(zero_v, acc.at[pl.ds(s * rpt + k * zb, zb)])
            return carry

        lax.fori_loop(0, rpt // zb, zcp, 0)
        plsc.subcore_barrier()

        def stage(j, p):
            off = s * ept + j * chunk
            pltpu.async_copy(src_hbm.at[pl.ds(off, chunk)], src_s[p], st_sem[p])
            pltpu.async_copy(dst_hbm.at[pl.ds(off, chunk)], idx_v[p].at[0],
                             st_sem[p])

        def prep(j, p):
            off = s * ept + j * chunk
            pltpu.make_async_copy(
                src_hbm.at[pl.ds(off, chunk)], src_s[p], st_sem[p]).wait()
            pltpu.make_async_copy(
                dst_hbm.at[pl.ds(off, chunk)], idx_v[p].at[0],
                st_sem[p]).wait()
            for v in range(chunk // 16):
                sl = pl.ds(v * 16, 16)
                src_s[p][sl] = src_s[p][sl] + cbase
            return pltpu.async_copy(tab_hbm.at[src_s[p]], rows_v[p], g_sem[p])

        def run_chunk(j, p):
            sd = pltpu.async_copy(rows_v[p], acc.at[idx_v[p].at[0]],
                                  sc_sem[p], add=True)
            gd = prep(jnp.minimum(j + 1, nchunk - 1), 1 - p)
            stage(jnp.minimum(j + 2, nchunk - 1), p)
            sd.wait()
            gd.wait()

        stage(0, 0)
        stage(1, 1)
        prep(0, 0).wait()

        def pair_body(i2, carry):
            run_chunk(2 * i2, 0)
            run_chunk(2 * i2 + 1, 1)
            return carry

        lax.fori_loop(0, nchunk // 2, pair_body, 0)
        plsc.subcore_barrier()
        pltpu.sync_copy(acc.at[pl.ds(s * rpt, rpt)],
                        out_hbm.at[pl.ds(c * n + s * rpt, rpt)])

    return segsum, ept * ns


def kernel(nodes, edges, features, W_node, b_node, W_conv1, b_conv1, W_out, b_out):
    n, fd = features.shape
    h = W_node.shape[1]
    hh = h // 2
    o = W_out.shape[1]
    e = edges.shape[0]

    nc, ns = 2, 16
    segsum, e_pad = _make_segsum(n, hh, e, nc, ns)

    src_p = jnp.concatenate(
        [edges[:, 0], jnp.zeros((e_pad - e,), jnp.int32)])
    dst_p = jnp.concatenate(
        [edges[:, 1], jnp.full((e_pad - e,), n, jnp.int32)])

    b_node2 = b_node.reshape(1, h)
    b_conv2 = b_conv1.reshape(1, h)
    b_out2 = b_out.reshape(1, o)

    hid = _mm1_split(features, W_node, b_node2)

    nh = segsum(hid.reshape(2 * n, hh), src_p, dst_p).reshape(2, n, hh)
    hid = _mm2_split(hid, nh, W_conv1, b_conv2)

    nh = segsum(hid.reshape(2 * n, hh), src_p, dst_p).reshape(2, n, hh)
    colsum = _mm2_colsum(hid, nh, W_conv1, b_conv2)

    metrics = _final_proj(colsum, W_out, b_out2, n)
    return metrics.reshape(o)


# speedup vs baseline: 9.3629x; 1.2644x over previous
import functools

import jax
import jax.numpy as jnp
from jax import lax
from jax.experimental import pallas as pl
from jax.experimental.pallas import tpu as pltpu
from jax.experimental.pallas import tpu_sc as plsc


def _mm1_split(x, w, b2):
    n, fd = x.shape
    h = w.shape[1]
    hh = h // 2
    bn = 2000

    def body(x_ref, w_ref, b_ref, o_ref):
        t = jnp.maximum(
            jnp.dot(x_ref[...], w_ref[...], preferred_element_type=jnp.float32)
            + b_ref[...],
            0.0,
        )
        o_ref[0] = t[:, :hh]
        o_ref[1] = t[:, hh:]

    return pl.pallas_call(
        body,
        grid=(n // bn,),
        in_specs=[
            pl.BlockSpec((bn, fd), lambda i: (i, 0)),
            pl.BlockSpec((fd, h), lambda i: (0, 0)),
            pl.BlockSpec((1, h), lambda i: (0, 0)),
        ],
        out_specs=pl.BlockSpec((2, bn, hh), lambda i: (0, i, 0)),
        out_shape=jax.ShapeDtypeStruct((2, n, hh), jnp.float32),
    )(x, w, b2)


def _mm2_split(a, nh, w, b2):
    _, n, hh = a.shape
    h = 2 * hh
    bn = 2000

    def body(a_ref, nh_ref, w_ref, b_ref, o_ref):
        x = jnp.concatenate(
            [a_ref[0] + nh_ref[0], a_ref[1] + nh_ref[1]], axis=1)
        t = jnp.maximum(
            jnp.dot(x, w_ref[...], preferred_element_type=jnp.float32)
            + b_ref[...],
            0.0,
        )
        o_ref[0] = t[:, :hh]
        o_ref[1] = t[:, hh:]

    return pl.pallas_call(
        body,
        grid=(n // bn,),
        in_specs=[
            pl.BlockSpec((2, bn, hh), lambda i: (0, i, 0)),
            pl.BlockSpec((2, bn, hh), lambda i: (0, i, 0)),
            pl.BlockSpec((h, h), lambda i: (0, 0)),
            pl.BlockSpec((1, h), lambda i: (0, 0)),
        ],
        out_specs=pl.BlockSpec((2, bn, hh), lambda i: (0, i, 0)),
        out_shape=jax.ShapeDtypeStruct((2, n, hh), jnp.float32),
    )(a, nh, w, b2)


def _mm2_colsum(a, nh, w, b2):
    _, n, hh = a.shape
    h = 2 * hh
    bn = 2000

    def body(a_ref, nh_ref, w_ref, b_ref, o_ref):
        x = jnp.concatenate(
            [a_ref[0] + nh_ref[0], a_ref[1] + nh_ref[1]], axis=1)
        t = jnp.maximum(
            jnp.dot(x, w_ref[...], preferred_element_type=jnp.float32)
            + b_ref[...],
            0.0,
        )
        s = jnp.sum(t, axis=0, keepdims=True)

        @pl.when(pl.program_id(0) == 0)
        def _():
            o_ref[...] = s

        @pl.when(pl.program_id(0) != 0)
        def _():
            o_ref[...] += s

    return pl.pallas_call(
        body,
        grid=(n // bn,),
        in_specs=[
            pl.BlockSpec((2, bn, hh), lambda i: (0, i, 0)),
            pl.BlockSpec((2, bn, hh), lambda i: (0, i, 0)),
            pl.BlockSpec((h, h), lambda i: (0, 0)),
            pl.BlockSpec((1, h), lambda i: (0, 0)),
        ],
        out_specs=pl.BlockSpec((1, h), lambda i: (0, 0)),
        out_shape=jax.ShapeDtypeStruct((1, h), jnp.float32),
    )(a, nh, w, b2)


def _final_proj(colsum, w_out, b2, n):
    def body(cs_ref, w_ref, b_ref, o_ref):
        o_ref[...] = (
            jnp.dot(cs_ref[...] * (1.0 / n), w_ref[...],
                    preferred_element_type=jnp.float32)
            + b_ref[...]
        )

    h, o = w_out.shape
    return pl.pallas_call(
        body,
        out_shape=jax.ShapeDtypeStruct((1, o), jnp.float32),
    )(colsum, w_out, b2)


def _round_up(x, m):
    return (x + m - 1) // m * m


def _make_segsum(n, hh, e, nc, ns):
    assert n % ns == 0
    rpt = n // ns
    chunk = 256
    ept = _round_up(-(-e // ns), 2 * chunk)
    nchunk = ept // chunk
    zb = 25
    assert rpt % zb == 0

    mesh = plsc.VectorSubcoreMesh(core_axis_name="c", subcore_axis_name="s")

    @functools.partial(
        pl.kernel,
        out_type=jax.ShapeDtypeStruct((nc * n, hh), jnp.float32),
        mesh=mesh,
        compiler_params=pltpu.CompilerParams(use_tc_tiling_on_sc=False),
        scratch_types=[
            [pltpu.VMEM((chunk,), jnp.int32)] * 2,
            [pltpu.VMEM((1, chunk), jnp.int32)] * 2,
            [pltpu.VMEM((chunk, hh), jnp.float32)] * 2,
            pltpu.VMEM((zb, hh), jnp.float32),
            pltpu.VMEM_SHARED((n + 8, hh), jnp.float32),
            [pltpu.SemaphoreType.DMA] * 2,
            [pltpu.SemaphoreType.DMA] * 2,
            [pltpu.SemaphoreType.DMA] * 2,
        ],
    )
    def segsum(tab_hbm, src_hbm, dst_hbm, out_hbm,
               src_s, idx_v, rows_v, zero_v, acc,
               st_sem, g_sem, sc_sem):
        c = lax.axis_index("c")
        s = lax.axis_index("s")
        tab_c = tab_hbm.at[c]

        def zrow(i, carry):
            for j in range(hh // 16):
                zero_v[i, pl.ds(j * 16, 16)] = jnp.zeros((16,), jnp.float32)
            return carry

        lax.fori_loop(0, zb, zrow, 0)

        def zcp(k, carry):
            pltpu.sync_copy(zero_v, acc.at[pl.ds(s * rpt + k * zb, zb)])
            return carry

        lax.fori_loop(0, rpt // zb, zcp, 0)
        plsc.subcore_barrier()

        def stage(j, p):
            off = s * ept + j * chunk
            pltpu.async_copy(src_hbm.at[pl.ds(off, chunk)], src_s[p], st_sem[p])
            pltpu.async_copy(dst_hbm.at[pl.ds(off, chunk)], idx_v[p].at[0],
                             st_sem[p])

        def prep(j, p):
            off = s * ept + j * chunk
            pltpu.make_async_copy(
                src_hbm.at[pl.ds(off, chunk)], src_s[p], st_sem[p]).wait()
            pltpu.make_async_copy(
                dst_hbm.at[pl.ds(off, chunk)], idx_v[p].at[0],
                st_sem[p]).wait()
            return pltpu.async_copy(tab_c.at[src_s[p]], rows_v[p], g_sem[p])

        def run_chunk(j, p):
            sd = pltpu.async_copy(rows_v[p], acc.at[idx_v[p].at[0]],
                                  sc_sem[p], add=True)
            gd = prep(jnp.minimum(j + 1, nchunk - 1), 1 - p)
            stage(jnp.minimum(j + 2, nchunk - 1), p)
            sd.wait()
            gd.wait()

        stage(0, 0)
        stage(1, 1)
        prep(0, 0).wait()

        def pair_body(i2, carry):
            run_chunk(2 * i2, 0)
            run_chunk(2 * i2 + 1, 1)
            return carry

        lax.fori_loop(0, nchunk // 2, pair_body, 0)
        plsc.subcore_barrier()
        pltpu.sync_copy(acc.at[pl.ds(s * rpt, rpt)],
                        out_hbm.at[pl.ds(c * n + s * rpt, rpt)])

    return segsum, ept * ns


def kernel(nodes, edges, features, W_node, b_node, W_conv1, b_conv1, W_out, b_out):
    n, fd = features.shape
    h = W_node.shape[1]
    hh = h // 2
    o = W_out.shape[1]
    e = edges.shape[0]

    nc, ns = 2, 16
    segsum, e_pad = _make_segsum(n, hh, e, nc, ns)

    src_p = jnp.concatenate(
        [edges[:, 0], jnp.zeros((e_pad - e,), jnp.int32)])
    dst_p = jnp.concatenate(
        [edges[:, 1], jnp.full((e_pad - e,), n, jnp.int32)])

    b_node2 = b_node.reshape(1, h)
    b_conv2 = b_conv1.reshape(1, h)
    b_out2 = b_out.reshape(1, o)

    hid = _mm1_split(features, W_node, b_node2)

    nh = segsum(hid, src_p, dst_p).reshape(2, n, hh)
    hid = _mm2_split(hid, nh, W_conv1, b_conv2)

    nh = segsum(hid, src_p, dst_p).reshape(2, n, hh)
    colsum = _mm2_colsum(hid, nh, W_conv1, b_conv2)

    metrics = _final_proj(colsum, W_out, b_out2, n)
    return metrics.reshape(o)


# speedup vs baseline: 9.3808x; 1.0019x over previous
import functools

import jax
import jax.numpy as jnp
from jax import lax
from jax.experimental import pallas as pl
from jax.experimental.pallas import tpu as pltpu
from jax.experimental.pallas import tpu_sc as plsc


def _mm1_split(x, w, b2):
    n, fd = x.shape
    h = w.shape[1]
    hh = h // 2
    bn = 2000

    def body(x_ref, w_ref, b_ref, o_ref):
        t = jnp.maximum(
            jnp.dot(x_ref[...], w_ref[...], preferred_element_type=jnp.float32)
            + b_ref[...],
            0.0,
        )
        o_ref[0] = t[:, :hh]
        o_ref[1] = t[:, hh:]

    return pl.pallas_call(
        body,
        grid=(n // bn,),
        in_specs=[
            pl.BlockSpec((bn, fd), lambda i: (i, 0)),
            pl.BlockSpec((fd, h), lambda i: (0, 0)),
            pl.BlockSpec((1, h), lambda i: (0, 0)),
        ],
        out_specs=pl.BlockSpec((2, bn, hh), lambda i: (0, i, 0)),
        out_shape=jax.ShapeDtypeStruct((2, n, hh), jnp.float32),
    )(x, w, b2)


def _mm2_split(a, nh, w, b2):
    _, n, hh = a.shape
    h = 2 * hh
    bn = 2000

    def body(a_ref, nh_ref, w_ref, b_ref, o_ref):
        x = jnp.concatenate(
            [a_ref[0] + nh_ref[0], a_ref[1] + nh_ref[1]], axis=1)
        t = jnp.maximum(
            jnp.dot(x, w_ref[...], preferred_element_type=jnp.float32)
            + b_ref[...],
            0.0,
        )
        o_ref[0] = t[:, :hh]
        o_ref[1] = t[:, hh:]

    return pl.pallas_call(
        body,
        grid=(n // bn,),
        in_specs=[
            pl.BlockSpec((2, bn, hh), lambda i: (0, i, 0)),
            pl.BlockSpec((2, bn, hh), lambda i: (0, i, 0)),
            pl.BlockSpec((h, h), lambda i: (0, 0)),
            pl.BlockSpec((1, h), lambda i: (0, 0)),
        ],
        out_specs=pl.BlockSpec((2, bn, hh), lambda i: (0, i, 0)),
        out_shape=jax.ShapeDtypeStruct((2, n, hh), jnp.float32),
    )(a, nh, w, b2)


def _mm2_metrics(a, nh, w, b2, w_out, bo2):
    _, n, hh = a.shape
    h = 2 * hh
    o = w_out.shape[1]
    bn = 2000
    nblk = n // bn

    def body(a_ref, nh_ref, w_ref, b_ref, wo_ref, bo_ref, o_ref, acc_ref):
        x = jnp.concatenate(
            [a_ref[0] + nh_ref[0], a_ref[1] + nh_ref[1]], axis=1)
        t = jnp.maximum(
            jnp.dot(x, w_ref[...], preferred_element_type=jnp.float32)
            + b_ref[...],
            0.0,
        )
        s = jnp.sum(t, axis=0, keepdims=True)

        @pl.when(pl.program_id(0) == 0)
        def _():
            acc_ref[...] = s

        @pl.when(pl.program_id(0) != 0)
        def _():
            acc_ref[...] += s

        @pl.when(pl.program_id(0) == nblk - 1)
        def _():
            o_ref[...] = (
                jnp.dot(acc_ref[...] * (1.0 / n), wo_ref[...],
                        preferred_element_type=jnp.float32)
                + bo_ref[...]
            )

    return pl.pallas_call(
        body,
        grid=(nblk,),
        in_specs=[
            pl.BlockSpec((2, bn, hh), lambda i: (0, i, 0)),
            pl.BlockSpec((2, bn, hh), lambda i: (0, i, 0)),
            pl.BlockSpec((h, h), lambda i: (0, 0)),
            pl.BlockSpec((1, h), lambda i: (0, 0)),
            pl.BlockSpec((h, o), lambda i: (0, 0)),
            pl.BlockSpec((1, o), lambda i: (0, 0)),
        ],
        out_specs=pl.BlockSpec((1, o), lambda i: (0, 0)),
        out_shape=jax.ShapeDtypeStruct((1, o), jnp.float32),
        scratch_shapes=[pltpu.VMEM((1, h), jnp.float32)],
    )(a, nh, w, b2, w_out, bo2)


def _round_up(x, m):
    return (x + m - 1) // m * m


def _make_segsum(n, hh, e, nc, ns):
    assert n % ns == 0
    rpt = n // ns
    chunk = 256
    ept = _round_up(-(-e // ns), 2 * chunk)
    nchunk = ept // chunk
    zb = 25
    assert rpt % zb == 0

    mesh = plsc.VectorSubcoreMesh(core_axis_name="c", subcore_axis_name="s")

    @functools.partial(
        pl.kernel,
        out_type=jax.ShapeDtypeStruct((nc * n, hh), jnp.float32),
        mesh=mesh,
        compiler_params=pltpu.CompilerParams(use_tc_tiling_on_sc=False),
        scratch_types=[
            [pltpu.VMEM((chunk,), jnp.int32)] * 2,
            [pltpu.VMEM((1, chunk), jnp.int32)] * 2,
            [pltpu.VMEM((chunk, hh), jnp.float32)] * 2,
            pltpu.VMEM((zb, hh), jnp.float32),
            pltpu.VMEM_SHARED((n + 8, hh), jnp.float32),
            [pltpu.SemaphoreType.DMA] * 2,
            [pltpu.SemaphoreType.DMA] * 2,
            [pltpu.SemaphoreType.DMA] * 2,
        ],
    )
    def segsum(tab_hbm, src_hbm, dst_hbm, out_hbm,
               src_s, idx_v, rows_v, zero_v, acc,
               st_sem, g_sem, sc_sem):
        c = lax.axis_index("c")
        s = lax.axis_index("s")
        tab_c = tab_hbm.at[c]

        def zrow(i, carry):
            for j in range(hh // 16):
                zero_v[i, pl.ds(j * 16, 16)] = jnp.zeros((16,), jnp.float32)
            return carry

        lax.fori_loop(0, zb, zrow, 0)

        def zcp(k, carry):
            pltpu.sync_copy(zero_v, acc.at[pl.ds(s * rpt + k * zb, zb)])
            return carry

        lax.fori_loop(0, rpt // zb, zcp, 0)
        plsc.subcore_barrier()

        def stage(j, p):
            off = s * ept + j * chunk
            pltpu.async_copy(src_hbm.at[pl.ds(off, chunk)], src_s[p], st_sem[p])
            pltpu.async_copy(dst_hbm.at[pl.ds(off, chunk)], idx_v[p].at[0],
                             st_sem[p])

        def prep(j, p):
            off = s * ept + j * chunk
            pltpu.make_async_copy(
                src_hbm.at[pl.ds(off, chunk)], src_s[p], st_sem[p]).wait()
            pltpu.make_async_copy(
                dst_hbm.at[pl.ds(off, chunk)], idx_v[p].at[0],
                st_sem[p]).wait()
            return pltpu.async_copy(tab_c.at[src_s[p]], rows_v[p], g_sem[p])

        def run_chunk(j, p):
            sd = pltpu.async_copy(rows_v[p], acc.at[idx_v[p].at[0]],
                                  sc_sem[p], add=True)
            gd = prep(jnp.minimum(j + 1, nchunk - 1), 1 - p)
            stage(jnp.minimum(j + 2, nchunk - 1), p)
            sd.wait()
            gd.wait()

        stage(0, 0)
        stage(1, 1)
        prep(0, 0).wait()

        def pair_body(i2, carry):
            run_chunk(2 * i2, 0)
            run_chunk(2 * i2 + 1, 1)
            return carry

        lax.fori_loop(0, nchunk // 2, pair_body, 0)
        plsc.subcore_barrier()
        pltpu.sync_copy(acc.at[pl.ds(s * rpt, rpt)],
                        out_hbm.at[pl.ds(c * n + s * rpt, rpt)])

    return segsum, ept * ns


def kernel(nodes, edges, features, W_node, b_node, W_conv1, b_conv1, W_out, b_out):
    n, fd = features.shape
    h = W_node.shape[1]
    hh = h // 2
    o = W_out.shape[1]
    e = edges.shape[0]

    nc, ns = 2, 16
    segsum, e_pad = _make_segsum(n, hh, e, nc, ns)

    src_p = jnp.concatenate(
        [edges[:, 0], jnp.zeros((e_pad - e,), jnp.int32)])
    dst_p = jnp.concatenate(
        [edges[:, 1], jnp.full((e_pad - e,), n, jnp.int32)])

    b_node2 = b_node.reshape(1, h)
    b_conv2 = b_conv1.reshape(1, h)
    b_out2 = b_out.reshape(1, o)

    hid = _mm1_split(features, W_node, b_node2)

    nh = segsum(hid, src_p, dst_p).reshape(2, n, hh)
    hid = _mm2_split(hid, nh, W_conv1, b_conv2)

    nh = segsum(hid, src_p, dst_p).reshape(2, n, hh)
    metrics = _mm2_metrics(hid, nh, W_conv1, b_conv2, W_out, b_out2)
    return metrics.reshape(o)


# speedup vs baseline: 9.5403x; 1.0170x over previous
import functools

import jax
import jax.numpy as jnp
from jax import lax
from jax.experimental import pallas as pl
from jax.experimental.pallas import tpu as pltpu
from jax.experimental.pallas import tpu_sc as plsc


def _mm1_split(x, w, b2):
    n, fd = x.shape
    h = w.shape[1]
    hh = h // 2
    bn = 2000

    def body(x_ref, w_ref, b_ref, o_ref):
        t = jnp.maximum(
            jnp.dot(x_ref[...], w_ref[...], preferred_element_type=jnp.float32)
            + b_ref[...],
            0.0,
        )
        o_ref[0] = t[:, :hh]
        o_ref[1] = t[:, hh:]

    return pl.pallas_call(
        body,
        grid=(n // bn,),
        in_specs=[
            pl.BlockSpec((bn, fd), lambda i: (i, 0)),
            pl.BlockSpec((fd, h), lambda i: (0, 0)),
            pl.BlockSpec((1, h), lambda i: (0, 0)),
        ],
        out_specs=pl.BlockSpec((2, bn, hh), lambda i: (0, i, 0)),
        out_shape=jax.ShapeDtypeStruct((2, n, hh), jnp.float32),
    )(x, w, b2)


def _mm2_split(a, nh, w, b2):
    _, n, hh = a.shape
    h = 2 * hh
    bn = 2000

    def body(a_ref, nh_ref, w_ref, b_ref, o_ref):
        x = jnp.concatenate(
            [a_ref[0] + nh_ref[0], a_ref[1] + nh_ref[1]], axis=1)
        t = jnp.maximum(
            jnp.dot(x, w_ref[...], preferred_element_type=jnp.float32)
            + b_ref[...],
            0.0,
        )
        o_ref[0] = t[:, :hh]
        o_ref[1] = t[:, hh:]

    return pl.pallas_call(
        body,
        grid=(n // bn,),
        in_specs=[
            pl.BlockSpec((2, bn, hh), lambda i: (0, i, 0)),
            pl.BlockSpec((2, bn, hh), lambda i: (0, i, 0)),
            pl.BlockSpec((h, h), lambda i: (0, 0)),
            pl.BlockSpec((1, h), lambda i: (0, 0)),
        ],
        out_specs=pl.BlockSpec((2, bn, hh), lambda i: (0, i, 0)),
        out_shape=jax.ShapeDtypeStruct((2, n, hh), jnp.float32),
    )(a, nh, w, b2)


def _mm2_metrics(a, nh, w, b2, w_out, bo2):
    _, n, hh = a.shape
    h = 2 * hh
    o = w_out.shape[1]
    bn = 2000
    nblk = n // bn

    def body(a_ref, nh_ref, w_ref, b_ref, wo_ref, bo_ref, o_ref, acc_ref):
        x = jnp.concatenate(
            [a_ref[0] + nh_ref[0], a_ref[1] + nh_ref[1]], axis=1)
        t = jnp.maximum(
            jnp.dot(x, w_ref[...], preferred_element_type=jnp.float32)
            + b_ref[...],
            0.0,
        )
        s = jnp.sum(t, axis=0, keepdims=True)

        @pl.when(pl.program_id(0) == 0)
        def _():
            acc_ref[...] = s

        @pl.when(pl.program_id(0) != 0)
        def _():
            acc_ref[...] += s

        @pl.when(pl.program_id(0) == nblk - 1)
        def _():
            o_ref[...] = (
                jnp.dot(acc_ref[...] * (1.0 / n), wo_ref[...],
                        preferred_element_type=jnp.float32)
                + bo_ref[...]
            )

    return pl.pallas_call(
        body,
        grid=(nblk,),
        in_specs=[
            pl.BlockSpec((2, bn, hh), lambda i: (0, i, 0)),
            pl.BlockSpec((2, bn, hh), lambda i: (0, i, 0)),
            pl.BlockSpec((h, h), lambda i: (0, 0)),
            pl.BlockSpec((1, h), lambda i: (0, 0)),
            pl.BlockSpec((h, o), lambda i: (0, 0)),
            pl.BlockSpec((1, o), lambda i: (0, 0)),
        ],
        out_specs=pl.BlockSpec((1, o), lambda i: (0, 0)),
        out_shape=jax.ShapeDtypeStruct((1, o), jnp.float32),
        scratch_shapes=[pltpu.VMEM((1, h), jnp.float32)],
    )(a, nh, w, b2, w_out, bo2)


def _round_up(x, m):
    return (x + m - 1) // m * m


def _make_segsum(n, hh, e, nc, ns):
    assert n % ns == 0
    rpt = n // ns
    chunk = 256
    ept = _round_up(-(-e // ns), 2 * chunk)
    nchunk = ept // chunk
    zb = 125
    assert rpt % zb == 0

    mesh = plsc.VectorSubcoreMesh(core_axis_name="c", subcore_axis_name="s")

    @functools.partial(
        pl.kernel,
        out_type=jax.ShapeDtypeStruct((nc * n, hh), jnp.float32),
        mesh=mesh,
        compiler_params=pltpu.CompilerParams(use_tc_tiling_on_sc=False),
        scratch_types=[
            [pltpu.VMEM((chunk,), jnp.int32)] * 2,
            [pltpu.VMEM((1, chunk), jnp.int32)] * 2,
            [pltpu.VMEM((chunk, hh), jnp.float32)] * 2,
            pltpu.VMEM((zb, hh), jnp.float32),
            pltpu.VMEM_SHARED((n + 8, hh), jnp.float32),
            [pltpu.SemaphoreType.DMA] * 2,
            [pltpu.SemaphoreType.DMA] * 2,
            [pltpu.SemaphoreType.DMA] * 2,
            pltpu.SemaphoreType.DMA,
        ],
    )
    def segsum(tab_hbm, src_hbm, dst_hbm, out_hbm,
               src_s, idx_v, rows_v, zero_v, acc,
               st_sem, g_sem, sc_sem, z_sem):
        c = lax.axis_index("c")
        s = lax.axis_index("s")
        tab_c = tab_hbm.at[c]

        def zrow(i, carry):
            for j in range(hh // 16):
                zero_v[i, pl.ds(j * 16, 16)] = jnp.zeros((16,), jnp.float32)
            return carry

        lax.fori_loop(0, zb, zrow, 0)

        zd = [
            pltpu.async_copy(zero_v, acc.at[pl.ds(s * rpt + k * zb, zb)],
                             z_sem)
            for k in range(rpt // zb)
        ]

        def stage(j, p):
            off = s * ept + j * chunk
            pltpu.async_copy(src_hbm.at[pl.ds(off, chunk)], src_s[p], st_sem[p])
            pltpu.async_copy(dst_hbm.at[pl.ds(off, chunk)], idx_v[p].at[0],
                             st_sem[p])

        def prep(j, p):
            off = s * ept + j * chunk
            pltpu.make_async_copy(
                src_hbm.at[pl.ds(off, chunk)], src_s[p], st_sem[p]).wait()
            pltpu.make_async_copy(
                dst_hbm.at[pl.ds(off, chunk)], idx_v[p].at[0],
                st_sem[p]).wait()
            return pltpu.async_copy(tab_c.at[src_s[p]], rows_v[p], g_sem[p])

        def run_chunk(j, p):
            sd = pltpu.async_copy(rows_v[p], acc.at[idx_v[p].at[0]],
                                  sc_sem[p], add=True)
            gd = prep(jnp.minimum(j + 1, nchunk - 1), 1 - p)
            stage(jnp.minimum(j + 2, nchunk - 1), p)
            sd.wait()
            gd.wait()

        stage(0, 0)
        stage(1, 1)
        gd0 = prep(0, 0)
        for d in zd:
            d.wait()
        plsc.subcore_barrier()
        gd0.wait()

        def pair_body(i2, carry):
            run_chunk(2 * i2, 0)
            run_chunk(2 * i2 + 1, 1)
            return carry

        lax.fori_loop(0, nchunk // 2, pair_body, 0)
        plsc.subcore_barrier()
        pltpu.sync_copy(acc.at[pl.ds(s * rpt, rpt)],
                        out_hbm.at[pl.ds(c * n + s * rpt, rpt)])

    return segsum, ept * ns


def kernel(nodes, edges, features, W_node, b_node, W_conv1, b_conv1, W_out, b_out):
    n, fd = features.shape
    h = W_node.shape[1]
    hh = h // 2
    o = W_out.shape[1]
    e = edges.shape[0]

    nc, ns = 2, 16
    segsum, e_pad = _make_segsum(n, hh, e, nc, ns)

    src_p = jnp.concatenate(
        [edges[:, 0], jnp.zeros((e_pad - e,), jnp.int32)])
    dst_p = jnp.concatenate(
        [edges[:, 1], jnp.full((e_pad - e,), n, jnp.int32)])

    b_node2 = b_node.reshape(1, h)
    b_conv2 = b_conv1.reshape(1, h)
    b_out2 = b_out.reshape(1, o)

    hid = _mm1_split(features, W_node, b_node2)

    nh = segsum(hid, src_p, dst_p).reshape(2, n, hh)
    hid = _mm2_split(hid, nh, W_conv1, b_conv2)

    nh = segsum(hid, src_p, dst_p).reshape(2, n, hh)
    metrics = _mm2_metrics(hid, nh, W_conv1, b_conv2, W_out, b_out2)
    return metrics.reshape(o)


# speedup vs baseline: 10.1606x; 1.0650x over previous
import functools

import jax
import jax.numpy as jnp
from jax import lax
from jax.experimental import pallas as pl
from jax.experimental.pallas import tpu as pltpu
from jax.experimental.pallas import tpu_sc as plsc


def _mm1_split(x, w, b2):
    n, fd = x.shape
    h = w.shape[1]
    hh = h // 2
    bn = 2000

    def body(x_ref, w_ref, b_ref, o_ref):
        t = jnp.maximum(
            jnp.dot(x_ref[...], w_ref[...], preferred_element_type=jnp.float32)
            + b_ref[...],
            0.0,
        )
        o_ref[0] = t[:, :hh]
        o_ref[1] = t[:, hh:]

    return pl.pallas_call(
        body,
        grid=(n // bn,),
        in_specs=[
            pl.BlockSpec((bn, fd), lambda i: (i, 0)),
            pl.BlockSpec((fd, h), lambda i: (0, 0)),
            pl.BlockSpec((1, h), lambda i: (0, 0)),
        ],
        out_specs=pl.BlockSpec((2, bn, hh), lambda i: (0, i, 0)),
        out_shape=jax.ShapeDtypeStruct((2, n, hh), jnp.float32),
    )(x, w, b2)


def _mm2_split(a, nh, w, b2):
    _, n, hh = a.shape
    h = 2 * hh
    bn = 2000

    def body(a_ref, nh_ref, w_ref, b_ref, o_ref):
        x = jnp.concatenate(
            [a_ref[0] + nh_ref[0], a_ref[1] + nh_ref[1]], axis=1)
        t = jnp.maximum(
            jnp.dot(x, w_ref[...], preferred_element_type=jnp.float32)
            + b_ref[...],
            0.0,
        )
        o_ref[0] = t[:, :hh]
        o_ref[1] = t[:, hh:]

    return pl.pallas_call(
        body,
        grid=(n // bn,),
        in_specs=[
            pl.BlockSpec((2, bn, hh), lambda i: (0, i, 0)),
            pl.BlockSpec((2, bn, hh), lambda i: (0, i, 0)),
            pl.BlockSpec((h, h), lambda i: (0, 0)),
            pl.BlockSpec((1, h), lambda i: (0, 0)),
        ],
        out_specs=pl.BlockSpec((2, bn, hh), lambda i: (0, i, 0)),
        out_shape=jax.ShapeDtypeStruct((2, n, hh), jnp.float32),
    )(a, nh, w, b2)


def _mm2_metrics(a, nh, w, b2, w_out, bo2):
    _, n, hh = a.shape
    h = 2 * hh
    o = w_out.shape[1]
    bn = 2000
    nblk = n // bn

    def body(a_ref, nh_ref, w_ref, b_ref, wo_ref, bo_ref, o_ref, acc_ref):
        x = jnp.concatenate(
            [a_ref[0] + nh_ref[0], a_ref[1] + nh_ref[1]], axis=1)
        t = jnp.maximum(
            jnp.dot(x, w_ref[...], preferred_element_type=jnp.float32)
            + b_ref[...],
            0.0,
        )
        s = jnp.sum(t, axis=0, keepdims=True)

        @pl.when(pl.program_id(0) == 0)
        def _():
            acc_ref[...] = s

        @pl.when(pl.program_id(0) != 0)
        def _():
            acc_ref[...] += s

        @pl.when(pl.program_id(0) == nblk - 1)
        def _():
            o_ref[...] = (
                jnp.dot(acc_ref[...] * (1.0 / n), wo_ref[...],
                        preferred_element_type=jnp.float32)
                + bo_ref[...]
            )

    return pl.pallas_call(
        body,
        grid=(nblk,),
        in_specs=[
            pl.BlockSpec((2, bn, hh), lambda i: (0, i, 0)),
            pl.BlockSpec((2, bn, hh), lambda i: (0, i, 0)),
            pl.BlockSpec((h, h), lambda i: (0, 0)),
            pl.BlockSpec((1, h), lambda i: (0, 0)),
            pl.BlockSpec((h, o), lambda i: (0, 0)),
            pl.BlockSpec((1, o), lambda i: (0, 0)),
        ],
        out_specs=pl.BlockSpec((1, o), lambda i: (0, 0)),
        out_shape=jax.ShapeDtypeStruct((1, o), jnp.float32),
        scratch_shapes=[pltpu.VMEM((1, h), jnp.float32)],
    )(a, nh, w, b2, w_out, bo2)


def _round_up(x, m):
    return (x + m - 1) // m * m


def _make_segsum(n, hh, e, nc, ns):
    assert n % ns == 0
    rpt = n // ns
    chunk = 288
    ept = _round_up(-(-e // ns), 2 * chunk)
    nchunk = ept // chunk
    zb = 125
    assert rpt % zb == 0

    mesh = plsc.VectorSubcoreMesh(core_axis_name="c", subcore_axis_name="s")

    @functools.partial(
        pl.kernel,
        out_type=jax.ShapeDtypeStruct((nc * n, hh), jnp.float32),
        mesh=mesh,
        compiler_params=pltpu.CompilerParams(use_tc_tiling_on_sc=False),
        scratch_types=[
            [pltpu.VMEM((chunk,), jnp.int32)] * 2,
            [pltpu.VMEM((1, chunk), jnp.int32)] * 2,
            [pltpu.VMEM((chunk, hh), jnp.float32)] * 2,
            pltpu.VMEM((zb, hh), jnp.float32),
            pltpu.VMEM_SHARED((n + 8, hh), jnp.float32),
            [pltpu.SemaphoreType.DMA] * 2,
            [pltpu.SemaphoreType.DMA] * 2,
            [pltpu.SemaphoreType.DMA] * 2,
            pltpu.SemaphoreType.DMA,
        ],
    )
    def segsum(tab_hbm, src_hbm, dst_hbm, out_hbm,
               src_s, idx_v, rows_v, zero_v, acc,
               st_sem, g_sem, sc_sem, z_sem):
        c = lax.axis_index("c")
        s = lax.axis_index("s")
        tab_c = tab_hbm.at[c]

        def zrow(i, carry):
            for j in range(hh // 16):
                zero_v[i, pl.ds(j * 16, 16)] = jnp.zeros((16,), jnp.float32)
            return carry

        lax.fori_loop(0, zb, zrow, 0)

        zd = [
            pltpu.async_copy(zero_v, acc.at[pl.ds(s * rpt + k * zb, zb)],
                             z_sem)
            for k in range(rpt // zb)
        ]

        def stage(j, p):
            off = s * ept + j * chunk
            pltpu.async_copy(src_hbm.at[pl.ds(off, chunk)], src_s[p], st_sem[p])
            pltpu.async_copy(dst_hbm.at[pl.ds(off, chunk)], idx_v[p].at[0],
                             st_sem[p])

        def prep(j, p):
            off = s * ept + j * chunk
            pltpu.make_async_copy(
                src_hbm.at[pl.ds(off, chunk)], src_s[p], st_sem[p]).wait()
            pltpu.make_async_copy(
                dst_hbm.at[pl.ds(off, chunk)], idx_v[p].at[0],
                st_sem[p]).wait()
            return pltpu.async_copy(tab_c.at[src_s[p]], rows_v[p], g_sem[p])

        def run_chunk(j, p):
            sd = pltpu.async_copy(rows_v[p], acc.at[idx_v[p].at[0]],
                                  sc_sem[p], add=True)
            gd = prep(jnp.minimum(j + 1, nchunk - 1), 1 - p)
            stage(jnp.minimum(j + 2, nchunk - 1), p)
            sd.wait()
            gd.wait()

        stage(0, 0)
        stage(1, 1)
        gd0 = prep(0, 0)
        for d in zd:
            d.wait()
        plsc.subcore_barrier()
        gd0.wait()

        def pair_body(i2, carry):
            run_chunk(2 * i2, 0)
            run_chunk(2 * i2 + 1, 1)
            return carry

        lax.fori_loop(0, nchunk // 2, pair_body, 0)
        plsc.subcore_barrier()
        pltpu.sync_copy(acc.at[pl.ds(s * rpt, rpt)],
                        out_hbm.at[pl.ds(c * n + s * rpt, rpt)])

    return segsum, ept * ns


def kernel(nodes, edges, features, W_node, b_node, W_conv1, b_conv1, W_out, b_out):
    n, fd = features.shape
    h = W_node.shape[1]
    hh = h // 2
    o = W_out.shape[1]
    e = edges.shape[0]

    nc, ns = 2, 16
    segsum, e_pad = _make_segsum(n, hh, e, nc, ns)

    src_p = jnp.concatenate(
        [edges[:, 0], jnp.zeros((e_pad - e,), jnp.int32)])
    dst_p = jnp.concatenate(
        [edges[:, 1], jnp.full((e_pad - e,), n, jnp.int32)])

    b_node2 = b_node.reshape(1, h)
    b_conv2 = b_conv1.reshape(1, h)
    b_out2 = b_out.reshape(1, o)

    hid = _mm1_split(features, W_node, b_node2)

    nh = segsum(hid, src_p, dst_p).reshape(2, n, hh)
    hid = _mm2_split(hid, nh, W_conv1, b_conv2)

    nh = segsum(hid, src_p, dst_p).reshape(2, n, hh)
    metrics = _mm2_metrics(hid, nh, W_conv1, b_conv2, W_out, b_out2)
    return metrics.reshape(o)
